# baseline (device time: 16046 ns/iter reference)
import jax
import jax.numpy as jnp
from jax import lax
from jax.experimental import pallas as pl
from jax.experimental.pallas import tpu as pltpu

N_DEV = 4


def kernel(x):
    m_per, n = x.shape
    m_half = m_per // 2

    def body(x_ref, out_ref, gather, send_sems, recv_sems, local_sems):
        my_pos = lax.axis_index("i")
        left = (my_pos - 1) % N_DEV
        right = (my_pos + 1) % N_DEV
        diag = (my_pos + 2) % N_DEV

        def copy(rows, sem, dst):
            return pltpu.make_async_remote_copy(
                src_ref=gather.at[pl.ds(rows, m_half)],
                dst_ref=gather.at[pl.ds(rows, m_half)],
                send_sem=send_sems.at[sem],
                recv_sem=recv_sems.at[sem],
                device_id=(dst,),
                device_id_type=pl.DeviceIdType.MESH,
            )

        def flush(rows, size, sem):
            c = pltpu.make_async_copy(
                gather.at[pl.ds(rows, size)],
                out_ref.at[pl.ds(rows, size)],
                local_sems.at[sem],
            )
            c.start()
            return c

        barrier_sem = pltpu.get_barrier_semaphore()
        for nbr in [left, right]:
            pl.semaphore_signal(
                barrier_sem, inc=1,
                device_id=(nbr,), device_id_type=pl.DeviceIdType.MESH,
            )
        pl.semaphore_wait(barrier_sem, 2)

        gather[pl.ds(my_pos * m_per, m_per), :] = x_ref[:, :].astype(
            jnp.bfloat16
        )

        r1 = [
            copy(my_pos * m_per, 0, right),
            copy(my_pos * m_per + m_half, 1, right),
            copy(my_pos * m_per + m_half, 2, left),
            copy(my_pos * m_per, 3, left),
        ]
        for r in r1:
            r.start()
        flushes = [flush(my_pos * m_per, m_per, 0)]

        copy(left * m_per, 0, right).wait_recv()
        fwd_right = copy(left * m_per, 4, right)
        fwd_right.start()
        flushes.append(flush(left * m_per, m_half, 1))

        copy(right * m_per + m_half, 2, left).wait_recv()
        fwd_left = copy(right * m_per + m_half, 5, left)
        fwd_left.start()
        flushes.append(flush(right * m_per + m_half, m_half, 2))

        copy(left * m_per + m_half, 1, right).wait_recv()
        flushes.append(flush(left * m_per + m_half, m_half, 3))
        copy(right * m_per, 3, left).wait_recv()
        flushes.append(flush(right * m_per, m_half, 4))
        copy(diag * m_per, 4, right).wait_recv()
        flushes.append(flush(diag * m_per, m_half, 5))
        copy(diag * m_per + m_half, 5, left).wait_recv()
        flushes.append(flush(diag * m_per + m_half, m_half, 6))

        for r in r1:
            r.wait_send()
        fwd_right.wait_send()
        fwd_left.wait_send()
        for c in flushes:
            c.wait()

    return pl.pallas_call(
        body,
        out_shape=jax.ShapeDtypeStruct((N_DEV * m_per, n), jnp.bfloat16),
        in_specs=[pl.BlockSpec(memory_space=pltpu.VMEM)],
        out_specs=pl.BlockSpec(memory_space=pl.ANY),
        scratch_shapes=[
            pltpu.VMEM((N_DEV * m_per, n), jnp.bfloat16),
            pltpu.SemaphoreType.DMA((6,)),
            pltpu.SemaphoreType.DMA((6,)),
            pltpu.SemaphoreType.DMA((7,)),
        ],
        compiler_params=pltpu.CompilerParams(collective_id=0),
    )(x)


# device time: 15899 ns/iter; 1.0092x vs baseline; 1.0092x over previous
import jax
import jax.numpy as jnp
from jax import lax
from jax.experimental import pallas as pl
from jax.experimental.pallas import tpu as pltpu

N_DEV = 4


def kernel(x):
    m_per, n = x.shape
    m_half = m_per // 2

    def body(x_ref, out_ref, send_sems, recv_sems):
        my_pos = lax.axis_index("i")
        left = (my_pos - 1) % N_DEV
        right = (my_pos + 1) % N_DEV
        diag = (my_pos + 2) % N_DEV

        def copy(rows, sem, dst):
            return pltpu.make_async_remote_copy(
                src_ref=out_ref.at[pl.ds(rows, m_half)],
                dst_ref=out_ref.at[pl.ds(rows, m_half)],
                send_sem=send_sems.at[sem],
                recv_sem=recv_sems.at[sem],
                device_id=(dst,),
                device_id_type=pl.DeviceIdType.MESH,
            )

        barrier_sem = pltpu.get_barrier_semaphore()
        for nbr in [left, right]:
            pl.semaphore_signal(
                barrier_sem, inc=1,
                device_id=(nbr,), device_id_type=pl.DeviceIdType.MESH,
            )
        pl.semaphore_wait(barrier_sem, 2)

        out_ref[pl.ds(my_pos * m_per, m_half), :] = x_ref[
            pl.ds(0, m_half), :
        ].astype(jnp.bfloat16)
        r1 = [copy(my_pos * m_per, 0, right)]
        r1[0].start()
        out_ref[pl.ds(my_pos * m_per + m_half, m_half), :] = x_ref[
            pl.ds(m_half, m_half), :
        ].astype(jnp.bfloat16)
        r1 += [
            copy(my_pos * m_per + m_half, 2, left),
            copy(my_pos * m_per + m_half, 1, right),
            copy(my_pos * m_per, 3, left),
        ]
        for r in r1[1:]:
            r.start()

        from_left_top = copy(left * m_per, 0, right)
        from_left_top.wait_recv()
        fwd_right = copy(left * m_per, 4, right)
        fwd_right.start()

        from_right_bot = copy(right * m_per + m_half, 2, left)
        from_right_bot.wait_recv()
        fwd_left = copy(right * m_per + m_half, 5, left)
        fwd_left.start()

        copy(left * m_per + m_half, 1, right).wait_recv()
        copy(right * m_per, 3, left).wait_recv()
        copy(diag * m_per, 4, right).wait_recv()
        copy(diag * m_per + m_half, 5, left).wait_recv()

        for r in r1:
            r.wait_send()
        fwd_right.wait_send()
        fwd_left.wait_send()

    return pl.pallas_call(
        body,
        out_shape=jax.ShapeDtypeStruct((N_DEV * m_per, n), jnp.bfloat16),
        in_specs=[pl.BlockSpec(memory_space=pltpu.VMEM)],
        out_specs=pl.BlockSpec(memory_space=pltpu.VMEM),
        scratch_shapes=[
            pltpu.SemaphoreType.DMA((6,)),
            pltpu.SemaphoreType.DMA((6,)),
        ],
        compiler_params=pltpu.CompilerParams(collective_id=0),
    )(x)


# device time: 15882 ns/iter; 1.0103x vs baseline; 1.0011x over previous
import jax
import jax.numpy as jnp
from jax import lax
from jax.experimental import pallas as pl
from jax.experimental.pallas import tpu as pltpu

N_DEV = 4


def kernel(x):
    m_per, n = x.shape
    m_half = m_per // 2

    def body(x_ref, out_ref, send_sems, recv_sems):
        my_pos = lax.axis_index("i")
        left = (my_pos - 1) % N_DEV
        right = (my_pos + 1) % N_DEV
        diag = (my_pos + 2) % N_DEV

        def copy(rows, sem, dst):
            return pltpu.make_async_remote_copy(
                src_ref=out_ref.at[pl.ds(rows, m_half)],
                dst_ref=out_ref.at[pl.ds(rows, m_half)],
                send_sem=send_sems.at[sem],
                recv_sem=recv_sems.at[sem],
                device_id=(dst,),
                device_id_type=pl.DeviceIdType.MESH,
            )

        barrier_sem = pltpu.get_barrier_semaphore()
        for nbr in [left, right]:
            pl.semaphore_signal(
                barrier_sem, inc=1,
                device_id=(nbr,), device_id_type=pl.DeviceIdType.MESH,
            )

        out_ref[pl.ds(my_pos * m_per, m_half), :] = x_ref[
            pl.ds(0, m_half), :
        ].astype(jnp.bfloat16)
        pl.semaphore_wait(barrier_sem, 2)
        r1 = [copy(my_pos * m_per, 0, right)]
        r1[0].start()
        out_ref[pl.ds(my_pos * m_per + m_half, m_half), :] = x_ref[
            pl.ds(m_half, m_half), :
        ].astype(jnp.bfloat16)
        r1 += [
            copy(my_pos * m_per + m_half, 2, left),
            copy(my_pos * m_per + m_half, 1, right),
            copy(my_pos * m_per, 3, left),
        ]
        for r in r1[1:]:
            r.start()

        from_left_top = copy(left * m_per, 0, right)
        from_left_top.wait_recv()
        fwd_right = copy(left * m_per, 4, right)
        fwd_right.start()

        from_right_bot = copy(right * m_per + m_half, 2, left)
        from_right_bot.wait_recv()
        fwd_left = copy(right * m_per + m_half, 5, left)
        fwd_left.start()

        copy(left * m_per + m_half, 1, right).wait_recv()
        copy(right * m_per, 3, left).wait_recv()
        copy(diag * m_per, 4, right).wait_recv()
        copy(diag * m_per + m_half, 5, left).wait_recv()

        for r in r1:
            r.wait_send()
        fwd_right.wait_send()
        fwd_left.wait_send()

    return pl.pallas_call(
        body,
        out_shape=jax.ShapeDtypeStruct((N_DEV * m_per, n), jnp.bfloat16),
        in_specs=[pl.BlockSpec(memory_space=pltpu.VMEM)],
        out_specs=pl.BlockSpec(memory_space=pltpu.VMEM),
        scratch_shapes=[
            pltpu.SemaphoreType.DMA((6,)),
            pltpu.SemaphoreType.DMA((6,)),
        ],
        compiler_params=pltpu.CompilerParams(collective_id=0),
    )(x)
